# Initial kernel scaffold; baseline (speedup 1.0000x reference)
import jax
import jax.numpy as jnp
from jax.experimental import pallas as pl


def kernel(samples, keys_hv, level_hv, centroid_w):
    raise NotImplementedError("write your pallas kernel here")


# speedup vs baseline: 15.6878x; 15.6878x over previous
import jax
import jax.numpy as jnp
from jax.experimental import pallas as pl

_GROUP = 16


def _body(samples_ref, keys_ref, level_ref, cent_ref, out_ref):
    B, F = samples_ref.shape
    L, D = level_ref.shape
    x = (samples_ref[...] + 1.0) * (0.5 * (L - 1))
    idx = jnp.clip(jnp.round(x), 0, L - 1).astype(jnp.int32)
    level = level_ref[...].astype(jnp.bfloat16)
    l_iota = jax.lax.broadcasted_iota(jnp.int32, (1, _GROUP, L), 2)
    acc = jnp.zeros((B, D), jnp.float32)
    for g in range(F // _GROUP):
        keys_g = keys_ref[pl.ds(g * _GROUP, _GROUP), :].astype(jnp.bfloat16)
        m_g = (keys_g[:, None, :] * level[None, :, :]).reshape(_GROUP * L, D)
        idx_g = idx[:, g * _GROUP:(g + 1) * _GROUP]
        oh = (idx_g[:, :, None] == l_iota).astype(jnp.bfloat16).reshape(B, _GROUP * L)
        acc = acc + jnp.dot(oh, m_g, preferred_element_type=jnp.float32)
    enc = jnp.sign(acc)
    scores = jax.lax.dot_general(
        enc, cent_ref[...], (((1,), (1,)), ((), ())),
        preferred_element_type=jnp.float32)
    out_ref[...] = scores


def kernel(samples, keys_hv, level_hv, centroid_w):
    B = samples.shape[0]
    C = centroid_w.shape[0]
    return pl.pallas_call(
        _body,
        out_shape=jax.ShapeDtypeStruct((B, C), jnp.float32),
    )(samples, keys_hv, level_hv, centroid_w)
